# baseline (speedup 1.0000x reference)
import functools

import jax
import jax.numpy as jnp
from jax import lax
from jax.experimental import pallas as pl
from jax.experimental.pallas import tpu as pltpu
from jax.experimental.pallas import tpu_sc as plsc

_N = 10000
_E = 320000
_H = 128
_DE = 16

_NACC = 10240
_RPT = _NACC // 16
_CHUNK = 128
_NTILES = 32
_NCH = 80
_EPT = _NCH * _CHUNK
_EPAD = _NTILES * _EPT
_ITAIL = 8 * _CHUNK

def _deg_body(dst_hbm, zeros_hbm, ones_hbm, out, idx_v, ones_v, acc):
    cid = lax.axis_index("c")
    sid = lax.axis_index("s")
    sl = pl.ds(sid * _RPT, _RPT)
    pltpu.sync_copy(ones_hbm, ones_v)
    pltpu.sync_copy(zeros_hbm.at[sl], acc.at[sl])
    plsc.subcore_barrier()
    base = (cid * 16 + sid) * _EPT

    def body(i, c):
        off = base + i * _CHUNK
        pltpu.sync_copy(dst_hbm.at[pl.ds(off, _CHUNK)], idx_v)
        pltpu.sync_copy(ones_v, acc.at[idx_v], add=True)
        return c

    lax.fori_loop(0, _EPT // _CHUNK, body, 0)
    plsc.subcore_barrier()
    pltpu.sync_copy(acc.at[sl], out.at[pl.ds(cid * _NACC + sid * _RPT, _RPT)])


@functools.lru_cache(maxsize=None)
def _get_deg_kernel():
    return pl.kernel(
        _deg_body,
        out_type=jax.ShapeDtypeStruct((2 * _NACC,), jnp.float32),
        mesh=plsc.VectorSubcoreMesh(core_axis_name="c", subcore_axis_name="s"),
        scratch_types=[
            pltpu.VMEM((_CHUNK,), jnp.int32),
            pltpu.VMEM((_CHUNK,), jnp.float32),
            pltpu.VMEM_SHARED((_NACC,), jnp.float32),
        ],
    )


def _agg_body(y_hbm, init_hbm, src_hbm, dst_hbm, out,
              sidx0, didx0, rows0, sem0, sidx1, didx1, rows1, sem1, acc):
    cid = lax.axis_index("c")
    sid = lax.axis_index("s")
    sl = pl.ds(sid * _RPT, _RPT)
    wid = cid * 16 + sid
    pltpu.sync_copy(init_hbm.at[pl.ds(cid * _NACC + sid * _RPT, _RPT)],
                    acc.at[sl])
    plsc.subcore_barrier()
    base = wid * _EPT
    bufs = ((sidx0, didx0, rows0, sem0), (sidx1, didx1, rows1, sem1))

    def fire(bufset, i):
        sidx, didx, rows, sem = bufset
        ic = lax.min(i, _NCH - 1)
        off = base + ic * _CHUNK
        pltpu.sync_copy(src_hbm.at[pl.ds(off, _CHUNK)], sidx)
        pltpu.async_copy(y_hbm.at[sidx], rows, sem)
        pltpu.async_copy(dst_hbm.at[pl.ds(off, _CHUNK)], didx, sem)

    def drain(bufset):
        sidx, didx, rows, sem = bufset
        pltpu.make_async_copy(y_hbm.at[sidx], rows, sem).wait()
        pltpu.make_async_copy(dst_hbm.at[pl.ds(0, _CHUNK)], didx, sem).wait()

    for b in range(2):
        fire(bufs[b], b)

    def outer(j, c):
        for b in range(2):
            sidx, didx, rows, sem = bufs[b]
            i = 2 * j + b
            drain(bufs[b])
            pltpu.sync_copy(rows, acc.at[didx], add=True)
            fire(bufs[b], i + 2)
        return c

    lax.fori_loop(0, _NCH // 2, outer, 0)
    for b in range(2):
        drain(bufs[b])
    plsc.subcore_barrier()
    pltpu.sync_copy(acc.at[sl], out.at[pl.ds(cid * _NACC + sid * _RPT, _RPT)])


@functools.lru_cache(maxsize=None)
def _get_agg_kernel():
    return pl.kernel(
        _agg_body,
        out_type=jax.ShapeDtypeStruct((2 * _NACC, _H), jnp.float32),
        mesh=plsc.VectorSubcoreMesh(core_axis_name="c", subcore_axis_name="s"),
        scratch_types=[
            pltpu.VMEM((_CHUNK,), jnp.int32),
            pltpu.VMEM((_CHUNK,), jnp.int32),
            pltpu.VMEM((_CHUNK, _H), jnp.float32),
            pltpu.SemaphoreType.DMA,
            pltpu.VMEM((_CHUNK,), jnp.int32),
            pltpu.VMEM((_CHUNK,), jnp.int32),
            pltpu.VMEM((_CHUNK, _H), jnp.float32),
            pltpu.SemaphoreType.DMA,
            pltpu.VMEM_SHARED((_NACC, _H), jnp.float32),
        ],
    )


def _final_body(a_hbm, b_hbm, c_hbm, w2_hbm, src_hbm, dst_hbm, out_hbm,
                sidx0, didx0, ra0, rb0, rc0, sem0,
                sidx1, didx1, ra1, rb1, rc1, sem1, w2_v, o_v):
    cid = lax.axis_index("c")
    sid = lax.axis_index("s")
    wid = cid * 16 + sid
    base = wid * _EPT
    pltpu.sync_copy(w2_hbm, w2_v)
    bufs = ((sidx0, didx0, ra0, rb0, rc0, sem0),
            (sidx1, didx1, ra1, rb1, rc1, sem1))

    def fire(bufset, i):
        sidx, didx, ra, rb, rc, sem = bufset
        ic = lax.min(i, _NCH - 1)
        off = base + ic * _CHUNK
        pltpu.sync_copy(src_hbm.at[pl.ds(off, _CHUNK)], sidx)
        pltpu.sync_copy(dst_hbm.at[pl.ds(off, _CHUNK)], didx)
        pltpu.async_copy(a_hbm.at[sidx], ra, sem)
        pltpu.async_copy(b_hbm.at[didx], rb, sem)
        pltpu.async_copy(c_hbm.at[pl.ds(off, _CHUNK)], rc, sem)

    def drain(bufset):
        sidx, didx, ra, rb, rc, sem = bufset
        pltpu.make_async_copy(a_hbm.at[sidx], ra, sem).wait()
        pltpu.make_async_copy(b_hbm.at[didx], rb, sem).wait()
        pltpu.make_async_copy(c_hbm.at[pl.ds(0, _CHUNK)], rc, sem).wait()

    for b in range(2):
        fire(bufs[b], b)

    def outer(j, c):
        for b in range(2):
            sidx, didx, ra, rb, rc, sem = bufs[b]
            i = 2 * j + b
            drain(bufs[b])

            def ebody(e, c2):
                accv = jnp.zeros((16,), jnp.float32)
                for k in range(_H // 16):
                    s = pl.ds(k * 16, 16)
                    t = ra[e, s] + rb[e, s] + rc[e, s]
                    accv = accv + jnp.maximum(t, 0.0) * w2_v[s]
                o_v[e, :] = accv
                return c2

            lax.fori_loop(0, _CHUNK, ebody, 0)
            pltpu.sync_copy(o_v, out_hbm.at[pl.ds(base + i * _CHUNK, _CHUNK)])
            fire(bufs[b], i + 2)
        return c

    lax.fori_loop(0, _NCH // 2, outer, 0)
    for b in range(2):
        drain(bufs[b])


@functools.lru_cache(maxsize=None)
def _get_final_kernel():
    return pl.kernel(
        _final_body,
        out_type=jax.ShapeDtypeStruct((_EPAD, 16), jnp.float32),
        mesh=plsc.VectorSubcoreMesh(core_axis_name="c", subcore_axis_name="s"),
        scratch_types=[
            pltpu.VMEM((_CHUNK,), jnp.int32),
            pltpu.VMEM((_CHUNK,), jnp.int32),
            pltpu.VMEM((_CHUNK, _H), jnp.float32),
            pltpu.VMEM((_CHUNK, _H), jnp.float32),
            pltpu.VMEM((_CHUNK, _H), jnp.float32),
            pltpu.SemaphoreType.DMA,
            pltpu.VMEM((_CHUNK,), jnp.int32),
            pltpu.VMEM((_CHUNK,), jnp.int32),
            pltpu.VMEM((_CHUNK, _H), jnp.float32),
            pltpu.VMEM((_CHUNK, _H), jnp.float32),
            pltpu.VMEM((_CHUNK, _H), jnp.float32),
            pltpu.SemaphoreType.DMA,
            pltpu.VMEM((_H,), jnp.float32),
            pltpu.VMEM((_CHUNK, 16), jnp.float32),
        ],
    )


_BR = 512


def _k1_body(x_ref, w_ref, d0_ref, d1_ref, dinv_ref, y_ref):
    deg = d0_ref[:, 0] + d1_ref[:, 0] + 1.0
    dinv = lax.rsqrt(deg)
    dinv_ref[...] = dinv[:, None]
    xw = jnp.dot(x_ref[...], w_ref[...], preferred_element_type=jnp.float32)
    y_ref[...] = xw * dinv[:, None]


_NB = _NACC // _BR

_k1 = pl.pallas_call(
    _k1_body,
    grid=(_NB,),
    in_specs=[
        pl.BlockSpec((_BR, _H), lambda i: (i, 0)),
        pl.BlockSpec((_H, _H), lambda i: (0, 0)),
        pl.BlockSpec((_BR, 1), lambda i: (i, 0)),
        pl.BlockSpec((_BR, 1), lambda i: (i + _NB, 0)),
    ],
    out_specs=[
        pl.BlockSpec((_BR, 1), lambda i: (i, 0)),
        pl.BlockSpec((_BR, _H), lambda i: (i, 0)),
    ],
    out_shape=[
        jax.ShapeDtypeStruct((_NACC, 1), jnp.float32),
        jax.ShapeDtypeStruct((_NACC, _H), jnp.float32),
    ],
)


def _k2_body(p0_ref, p1_ref, dinv_ref, b_ref, w_ref, y_ref):
    dinv = dinv_ref[...]
    agg = p0_ref[...] + p1_ref[...]
    h = jnp.maximum(agg * dinv + b_ref[...], 0.0)
    y_ref[...] = jnp.dot(h, w_ref[...], preferred_element_type=jnp.float32) * dinv


_k2 = pl.pallas_call(
    _k2_body,
    grid=(_NB,),
    in_specs=[
        pl.BlockSpec((_BR, _H), lambda i: (i, 0)),
        pl.BlockSpec((_BR, _H), lambda i: (i + _NB, 0)),
        pl.BlockSpec((_BR, 1), lambda i: (i, 0)),
        pl.BlockSpec((1, _H), lambda i: (0, 0)),
        pl.BlockSpec((_H, _H), lambda i: (0, 0)),
    ],
    out_specs=pl.BlockSpec((_BR, _H), lambda i: (i, 0)),
    out_shape=jax.ShapeDtypeStruct((_NACC, _H), jnp.float32),
)


def _k3_body(p0_ref, p1_ref, dinv_ref, b_ref, wa_ref, wb_ref, a_ref, bo_ref):
    agg = p0_ref[...] + p1_ref[...]
    h = jnp.maximum(agg * dinv_ref[...] + b_ref[...], 0.0)
    a_ref[...] = jnp.dot(h, wa_ref[...], preferred_element_type=jnp.float32)
    bo_ref[...] = jnp.dot(h, wb_ref[...], preferred_element_type=jnp.float32)


_k3 = pl.pallas_call(
    _k3_body,
    grid=(_NB,),
    in_specs=[
        pl.BlockSpec((_BR, _H), lambda i: (i, 0)),
        pl.BlockSpec((_BR, _H), lambda i: (i + _NB, 0)),
        pl.BlockSpec((_BR, 1), lambda i: (i, 0)),
        pl.BlockSpec((1, _H), lambda i: (0, 0)),
        pl.BlockSpec((_H, _H), lambda i: (0, 0)),
        pl.BlockSpec((_H, _H), lambda i: (0, 0)),
    ],
    out_specs=[
        pl.BlockSpec((_BR, _H), lambda i: (i, 0)),
        pl.BlockSpec((_BR, _H), lambda i: (i, 0)),
    ],
    out_shape=[
        jax.ShapeDtypeStruct((_NACC, _H), jnp.float32),
        jax.ShapeDtypeStruct((_NACC, _H), jnp.float32),
    ],
)

_BE = 2048
_EPADC = _EPAD + _BE


def _k4_body(ea_ref, w_ref, b_ref, c_ref):
    c_ref[...] = jnp.dot(ea_ref[...], w_ref[...],
                         preferred_element_type=jnp.float32) + b_ref[...]


_k4 = pl.pallas_call(
    _k4_body,
    grid=(_EPADC // _BE,),
    in_specs=[
        pl.BlockSpec((_BE, _DE), lambda i: (i, 0)),
        pl.BlockSpec((_DE, _H), lambda i: (0, 0)),
        pl.BlockSpec((1, _H), lambda i: (0, 0)),
    ],
    out_specs=pl.BlockSpec((_BE, _H), lambda i: (i, 0)),
    out_shape=jax.ShapeDtypeStruct((_EPADC, _H), jnp.float32),
)


def _k5_body(p_ref, b_ref, o_ref):
    o_ref[...] = jnp.sum(p_ref[...], axis=1, keepdims=True) + b_ref[...]


_k5 = pl.pallas_call(
    _k5_body,
    grid=(_EPAD // _BE,),
    in_specs=[
        pl.BlockSpec((_BE, 16), lambda i: (i, 0)),
        pl.BlockSpec((1, 1), lambda i: (0, 0)),
    ],
    out_specs=pl.BlockSpec((_BE, 1), lambda i: (i, 0)),
    out_shape=jax.ShapeDtypeStruct((_EPAD, 1), jnp.float32),
)


def kernel(x, edge_index, edge_attr, W1, b1, W2, b2, W3, b3, Wf1, bf1, Wf2, bf2):
    pad = _EPAD - _E
    src = edge_index[0]
    dst = edge_index[1]
    srcp = jnp.concatenate([src, jnp.zeros((pad + _ITAIL,), jnp.int32)])
    dstp = jnp.concatenate(
        [dst, _N + (jnp.arange(pad, dtype=jnp.int32) % 128),
         jnp.zeros((_ITAIL,), jnp.int32)])
    xpad = jnp.pad(x, ((0, _NACC - _N), (0, 0)))
    eap = jnp.pad(edge_attr, ((0, _EPADC - _E), (0, 0)))
    zdeg = jnp.zeros((_NACC,), jnp.float32)
    ones_chunk = jnp.ones((_CHUNK,), jnp.float32)

    znodes = jnp.zeros((_NACC, _H), jnp.float32)

    dparts = _get_deg_kernel()(dstp, zdeg, ones_chunk).reshape(2 * _NACC, 1)
    dinv, y = _k1(xpad, W1, dparts, dparts)
    agg = _get_agg_kernel()
    p = agg(y, jnp.concatenate([y, znodes]), srcp, dstp)
    y2 = _k2(p, p, dinv, b1.reshape(1, -1), W2)
    p = agg(y2, jnp.concatenate([y2, znodes]), srcp, dstp)
    y3 = _k2(p, p, dinv, b2.reshape(1, -1), W3)
    p = agg(y3, jnp.concatenate([y3, znodes]), srcp, dstp)
    A, B = _k3(p, p, dinv, b3.reshape(1, -1),
               Wf1[:_H], Wf1[_H:2 * _H])
    C = _k4(eap, Wf1[2 * _H:], bf1.reshape(1, -1))
    partials = _get_final_kernel()(A, B, C, Wf2.reshape(-1), srcp, dstp)
    out = _k5(partials, bf2.reshape(1, 1))
    return out[:_E]


# speedup vs baseline: 1.2010x; 1.2010x over previous
import functools

import jax
import jax.numpy as jnp
from jax import lax
from jax.experimental import pallas as pl
from jax.experimental.pallas import tpu as pltpu
from jax.experimental.pallas import tpu_sc as plsc

_N = 10000
_E = 320000
_H = 128
_DE = 16

_NACC = 10240
_RPT = _NACC // 16
_CHUNK = 128
_NTILES = 32
_EPT = 79 * _CHUNK
_EPAD = _NTILES * _EPT


def _deg_body(dst_hbm, zeros_hbm, ones_hbm, out, idx_v, ones_v, acc):
    cid = lax.axis_index("c")
    sid = lax.axis_index("s")
    sl = pl.ds(sid * _RPT, _RPT)
    pltpu.sync_copy(ones_hbm, ones_v)
    pltpu.sync_copy(zeros_hbm.at[sl], acc.at[sl])
    plsc.subcore_barrier()
    base = (cid * 16 + sid) * _EPT

    def body(i, c):
        off = base + i * _CHUNK
        pltpu.sync_copy(dst_hbm.at[pl.ds(off, _CHUNK)], idx_v)
        pltpu.sync_copy(ones_v, acc.at[idx_v], add=True)
        return c

    lax.fori_loop(0, _EPT // _CHUNK, body, 0)
    plsc.subcore_barrier()
    pltpu.sync_copy(acc.at[sl], out.at[pl.ds(cid * _NACC + sid * _RPT, _RPT)])


@functools.lru_cache(maxsize=None)
def _get_deg_kernel():
    return pl.kernel(
        _deg_body,
        out_type=jax.ShapeDtypeStruct((2 * _NACC,), jnp.float32),
        mesh=plsc.VectorSubcoreMesh(core_axis_name="c", subcore_axis_name="s"),
        scratch_types=[
            pltpu.VMEM((_CHUNK,), jnp.int32),
            pltpu.VMEM((_CHUNK,), jnp.float32),
            pltpu.VMEM_SHARED((_NACC,), jnp.float32),
        ],
    )


def _agg_body(y_hbm, src_hbm, dst_hbm, out,
              sidx_v, didx_v, rows_v, acc, sem):
    cid = lax.axis_index("c")
    sid = lax.axis_index("s")
    sl = pl.ds(sid * _RPT, _RPT)
    pltpu.sync_copy(y_hbm.at[sl], acc.at[sl])
    plsc.subcore_barrier()
    base = (cid * 16 + sid) * _EPT

    def body(i, c):
        off = base + i * _CHUNK
        pltpu.sync_copy(src_hbm.at[pl.ds(off, _CHUNK)], sidx_v)
        pltpu.sync_copy(dst_hbm.at[pl.ds(off, _CHUNK)], didx_v)
        pltpu.async_copy(y_hbm.at[sidx_v], rows_v, sem).wait()
        pltpu.sync_copy(rows_v, acc.at[didx_v], add=True)
        return c

    lax.fori_loop(0, _EPT // _CHUNK, body, 0)
    plsc.subcore_barrier()
    pltpu.sync_copy(acc.at[sl], out.at[pl.ds(cid * _NACC + sid * _RPT, _RPT)])


@functools.lru_cache(maxsize=None)
def _get_agg_kernel():
    return pl.kernel(
        _agg_body,
        out_type=jax.ShapeDtypeStruct((2 * _NACC, _H), jnp.float32),
        mesh=plsc.VectorSubcoreMesh(core_axis_name="c", subcore_axis_name="s"),
        scratch_types=[
            pltpu.VMEM((_CHUNK,), jnp.int32),
            pltpu.VMEM((_CHUNK,), jnp.int32),
            pltpu.VMEM((_CHUNK, _H), jnp.float32),
            pltpu.VMEM_SHARED((_NACC, _H), jnp.float32),
            pltpu.SemaphoreType.DMA,
        ],
    )


def _final_body(a_hbm, b_hbm, c_hbm, w2_hbm, src_hbm, dst_hbm, out_hbm,
                sidx_v, didx_v, ra_v, rb_v, rc_v, w2_v, o_v, sem):
    cid = lax.axis_index("c")
    sid = lax.axis_index("s")
    pltpu.sync_copy(w2_hbm, w2_v)
    base = (cid * 16 + sid) * _EPT

    def body(i, c):
        off = base + i * _CHUNK
        pltpu.sync_copy(src_hbm.at[pl.ds(off, _CHUNK)], sidx_v)
        pltpu.sync_copy(dst_hbm.at[pl.ds(off, _CHUNK)], didx_v)
        pltpu.async_copy(a_hbm.at[sidx_v], ra_v, sem).wait()
        pltpu.async_copy(b_hbm.at[didx_v], rb_v, sem).wait()
        pltpu.sync_copy(c_hbm.at[pl.ds(off, _CHUNK)], rc_v)

        def ebody(e, c2):
            accv = jnp.zeros((16,), jnp.float32)
            for k in range(_H // 16):
                s = pl.ds(k * 16, 16)
                t = ra_v[e, s] + rb_v[e, s] + rc_v[e, s]
                accv = accv + jnp.maximum(t, 0.0) * w2_v[s]
            o_v[e, :] = accv
            return c2

        lax.fori_loop(0, _CHUNK, ebody, 0)
        pltpu.sync_copy(o_v, out_hbm.at[pl.ds(off, _CHUNK)])
        return c

    lax.fori_loop(0, _EPT // _CHUNK, body, 0)


@functools.lru_cache(maxsize=None)
def _get_final_kernel():
    return pl.kernel(
        _final_body,
        out_type=jax.ShapeDtypeStruct((_EPAD, 16), jnp.float32),
        mesh=plsc.VectorSubcoreMesh(core_axis_name="c", subcore_axis_name="s"),
        scratch_types=[
            pltpu.VMEM((_CHUNK,), jnp.int32),
            pltpu.VMEM((_CHUNK,), jnp.int32),
            pltpu.VMEM((_CHUNK, _H), jnp.float32),
            pltpu.VMEM((_CHUNK, _H), jnp.float32),
            pltpu.VMEM((_CHUNK, _H), jnp.float32),
            pltpu.VMEM((_H,), jnp.float32),
            pltpu.VMEM((_CHUNK, 16), jnp.float32),
            pltpu.SemaphoreType.DMA,
        ],
    )


_BR = 512


def _k1_body(x_ref, w_ref, d0_ref, d1_ref, dinv_ref, y_ref):
    deg = d0_ref[:, 0] + d1_ref[:, 0] + 1.0
    dinv = lax.rsqrt(deg)
    dinv_ref[...] = dinv[:, None]
    xw = jnp.dot(x_ref[...], w_ref[...], preferred_element_type=jnp.float32)
    y_ref[...] = xw * dinv[:, None]


_NB = _NACC // _BR

_k1 = pl.pallas_call(
    _k1_body,
    grid=(_NB,),
    in_specs=[
        pl.BlockSpec((_BR, _H), lambda i: (i, 0)),
        pl.BlockSpec((_H, _H), lambda i: (0, 0)),
        pl.BlockSpec((_BR, 1), lambda i: (i, 0)),
        pl.BlockSpec((_BR, 1), lambda i: (i + _NB, 0)),
    ],
    out_specs=[
        pl.BlockSpec((_BR, 1), lambda i: (i, 0)),
        pl.BlockSpec((_BR, _H), lambda i: (i, 0)),
    ],
    out_shape=[
        jax.ShapeDtypeStruct((_NACC, 1), jnp.float32),
        jax.ShapeDtypeStruct((_NACC, _H), jnp.float32),
    ],
)


def _k2_body(p0_ref, p1_ref, yin_ref, dinv_ref, b_ref, w_ref, y_ref):
    dinv = dinv_ref[...]
    agg = p0_ref[...] + p1_ref[...] - yin_ref[...]
    h = jnp.maximum(agg * dinv + b_ref[...], 0.0)
    y_ref[...] = jnp.dot(h, w_ref[...], preferred_element_type=jnp.float32) * dinv


_k2 = pl.pallas_call(
    _k2_body,
    grid=(_NB,),
    in_specs=[
        pl.BlockSpec((_BR, _H), lambda i: (i, 0)),
        pl.BlockSpec((_BR, _H), lambda i: (i + _NB, 0)),
        pl.BlockSpec((_BR, _H), lambda i: (i, 0)),
        pl.BlockSpec((_BR, 1), lambda i: (i, 0)),
        pl.BlockSpec((1, _H), lambda i: (0, 0)),
        pl.BlockSpec((_H, _H), lambda i: (0, 0)),
    ],
    out_specs=pl.BlockSpec((_BR, _H), lambda i: (i, 0)),
    out_shape=jax.ShapeDtypeStruct((_NACC, _H), jnp.float32),
)


def _k3_body(p0_ref, p1_ref, yin_ref, dinv_ref, b_ref, wa_ref, wb_ref, a_ref, bo_ref):
    agg = p0_ref[...] + p1_ref[...] - yin_ref[...]
    h = jnp.maximum(agg * dinv_ref[...] + b_ref[...], 0.0)
    a_ref[...] = jnp.dot(h, wa_ref[...], preferred_element_type=jnp.float32)
    bo_ref[...] = jnp.dot(h, wb_ref[...], preferred_element_type=jnp.float32)


_k3 = pl.pallas_call(
    _k3_body,
    grid=(_NB,),
    in_specs=[
        pl.BlockSpec((_BR, _H), lambda i: (i, 0)),
        pl.BlockSpec((_BR, _H), lambda i: (i + _NB, 0)),
        pl.BlockSpec((_BR, _H), lambda i: (i, 0)),
        pl.BlockSpec((_BR, 1), lambda i: (i, 0)),
        pl.BlockSpec((1, _H), lambda i: (0, 0)),
        pl.BlockSpec((_H, _H), lambda i: (0, 0)),
        pl.BlockSpec((_H, _H), lambda i: (0, 0)),
    ],
    out_specs=[
        pl.BlockSpec((_BR, _H), lambda i: (i, 0)),
        pl.BlockSpec((_BR, _H), lambda i: (i, 0)),
    ],
    out_shape=[
        jax.ShapeDtypeStruct((_NACC, _H), jnp.float32),
        jax.ShapeDtypeStruct((_NACC, _H), jnp.float32),
    ],
)

_BE = 2048


def _k4_body(ea_ref, w_ref, b_ref, c_ref):
    c_ref[...] = jnp.dot(ea_ref[...], w_ref[...],
                         preferred_element_type=jnp.float32) + b_ref[...]


_k4 = pl.pallas_call(
    _k4_body,
    grid=(_EPAD // _BE,),
    in_specs=[
        pl.BlockSpec((_BE, _DE), lambda i: (i, 0)),
        pl.BlockSpec((_DE, _H), lambda i: (0, 0)),
        pl.BlockSpec((1, _H), lambda i: (0, 0)),
    ],
    out_specs=pl.BlockSpec((_BE, _H), lambda i: (i, 0)),
    out_shape=jax.ShapeDtypeStruct((_EPAD, _H), jnp.float32),
)


def _k5_body(p_ref, b_ref, o_ref):
    o_ref[...] = jnp.sum(p_ref[...], axis=1, keepdims=True) + b_ref[...]


_k5 = pl.pallas_call(
    _k5_body,
    grid=(_EPAD // _BE,),
    in_specs=[
        pl.BlockSpec((_BE, 16), lambda i: (i, 0)),
        pl.BlockSpec((1, 1), lambda i: (0, 0)),
    ],
    out_specs=pl.BlockSpec((_BE, 1), lambda i: (i, 0)),
    out_shape=jax.ShapeDtypeStruct((_EPAD, 1), jnp.float32),
)


def kernel(x, edge_index, edge_attr, W1, b1, W2, b2, W3, b3, Wf1, bf1, Wf2, bf2):
    pad = _EPAD - _E
    src = edge_index[0]
    dst = edge_index[1]
    srcp = jnp.concatenate([src, jnp.zeros((pad,), jnp.int32)])
    dstp = jnp.concatenate(
        [dst, _N + (jnp.arange(pad, dtype=jnp.int32) % 128)])
    xpad = jnp.pad(x, ((0, _NACC - _N), (0, 0)))
    eap = jnp.pad(edge_attr, ((0, pad), (0, 0)))
    zdeg = jnp.zeros((_NACC,), jnp.float32)
    ones_chunk = jnp.ones((_CHUNK,), jnp.float32)

    dparts = _get_deg_kernel()(dstp, zdeg, ones_chunk).reshape(2 * _NACC, 1)
    dinv, y = _k1(xpad, W1, dparts, dparts)
    agg = _get_agg_kernel()
    p = agg(y, srcp, dstp)
    y2 = _k2(p, p, y, dinv, b1.reshape(1, -1), W2)
    p = agg(y2, srcp, dstp)
    y3 = _k2(p, p, y2, dinv, b2.reshape(1, -1), W3)
    p = agg(y3, srcp, dstp)
    A, B = _k3(p, p, y3, dinv, b3.reshape(1, -1),
               Wf1[:_H], Wf1[_H:2 * _H])
    C = _k4(eap, Wf1[2 * _H:], bf1.reshape(1, -1))
    partials = _get_final_kernel()(A, B, C, Wf2.reshape(-1), srcp, dstp)
    out = _k5(partials, bf2.reshape(1, 1))
    return out[:_E]
